# baseline (speedup 1.0000x reference)
import functools
import numpy as np
import jax
import jax.numpy as jnp
from jax import lax
from jax.experimental import pallas as pl
from jax.experimental.pallas import tpu as pltpu
from jax.experimental.pallas import tpu_sc as plsc

N = 10000
E = 320000
C = 128
A = 4
S = 4
R = 8
H = 128
AVG_NUM_NEIGHBORS = 32.0
MLP_HIDDEN = 64

NC = 2
NS = 16
NW = NC * NS

K = 128
CH = 80
E_PER_W = CH * K
E_PAD = NW * E_PER_W

_inv_sqrt_C = np.float32(1.0 / np.sqrt(C))
_inv_sqrt_CA = np.float32(1.0 / np.sqrt(C * A))
_inv_sqrt_R = np.float32(1.0 / np.sqrt(R))
_inv_sqrt_M = np.float32(1.0 / np.sqrt(MLP_HIDDEN))
_coeff_scale = np.float32(1.0 / (np.sqrt(S) * np.sqrt(AVG_NUM_NEIGHBORS)))


def _silu(x):
    return x / (1.0 + jnp.exp(-x))


def _node_body(f_ref, attr_ref, w1_ref, wsc_ref, nf_ref, sc_ref):
    f = f_ref[...]
    nf_ref[...] = jnp.dot(f, w1_ref[...], preferred_element_type=jnp.float32) * _inv_sqrt_C
    acc = jnp.zeros_like(sc_ref)
    for a in range(A):
        fa = f * attr_ref[:, a][:, None]
        acc = acc + jnp.dot(fa, wsc_ref[a], preferred_element_type=jnp.float32)
    sc_ref[...] = acc * _inv_sqrt_CA


def _node_stage(node_feats, node_attrs, W1, W_sc):
    BN = 400
    grid = (N // BN,)
    wsc = W_sc.reshape(C, A, H).transpose(1, 0, 2)
    return pl.pallas_call(
        _node_body,
        grid=grid,
        in_specs=[
            pl.BlockSpec((BN, C), lambda i: (i, 0)),
            pl.BlockSpec((BN, A), lambda i: (i, 0)),
            pl.BlockSpec((C, C), lambda i: (0, 0)),
            pl.BlockSpec((A, C, H), lambda i: (0, 0, 0)),
        ],
        out_specs=[
            pl.BlockSpec((BN, C), lambda i: (i, 0)),
            pl.BlockSpec((BN, H), lambda i: (i, 0)),
        ],
        out_shape=[
            jax.ShapeDtypeStruct((N, C), jnp.float32),
            jax.ShapeDtypeStruct((N, H), jnp.float32),
        ],
    )(node_feats, node_attrs, W1, wsc)


def _edge_body(ef_ref, ea_ref, w1_ref, w2_ref, w3_ref, w4_ref, coeff_ref):
    h = _silu(jnp.dot(w1_ref[...], ef_ref[...], preferred_element_type=jnp.float32) * _inv_sqrt_R)
    h = _silu(jnp.dot(w2_ref[...], h, preferred_element_type=jnp.float32) * _inv_sqrt_M)
    h = _silu(jnp.dot(w3_ref[...], h, preferred_element_type=jnp.float32) * _inv_sqrt_M)
    lw = jnp.dot(w4_ref[...], h, preferred_element_type=jnp.float32) * _inv_sqrt_M
    coeff_ref[...] = jnp.sum(lw * ea_ref[...], axis=0) * _coeff_scale


def _edge_stage(edge_feats, edge_attrs, Wm1, Wm2, Wm3, Wm4):
    BE = 2048
    eft = jnp.zeros((R, E_PAD), jnp.float32).at[:, :E].set(edge_feats.T)
    eat = jnp.zeros((S, E_PAD), jnp.float32).at[:, :E].set(edge_attrs.T)
    grid = (E_PAD // BE,)
    return pl.pallas_call(
        _edge_body,
        grid=grid,
        in_specs=[
            pl.BlockSpec((R, BE), lambda i: (0, i)),
            pl.BlockSpec((S, BE), lambda i: (0, i)),
            pl.BlockSpec((MLP_HIDDEN, R), lambda i: (0, 0)),
            pl.BlockSpec((MLP_HIDDEN, MLP_HIDDEN), lambda i: (0, 0)),
            pl.BlockSpec((MLP_HIDDEN, MLP_HIDDEN), lambda i: (0, 0)),
            pl.BlockSpec((S, MLP_HIDDEN), lambda i: (0, 0)),
        ],
        out_specs=pl.BlockSpec((BE,), lambda i: (i,)),
        out_shape=jax.ShapeDtypeStruct((E_PAD,), jnp.float32),
    )(eft, eat, Wm1.T, Wm2.T, Wm3.T, Wm4.T)


N_PAD = 10240
N_PER_T = N_PAD // NS


def _sc_body(nf_hbm, s_hbm, r_hbm, c_hbm, out_hbm,
             sidx0, sidx1, ridx0, ridx1, cf_v, rows0, rows1, acc_sh,
             gsem0, gsem1, ssem0, ssem1, sisem0, sisem1, risem0, risem1):
    cid = lax.axis_index("c")
    sid = lax.axis_index("s")
    wid = sid * NC + cid

    def zrow(r, _):
        z16 = jnp.zeros((16,), jnp.float32)
        for t in range(C // 16):
            rows0[r, pl.ds(t * 16, 16)] = z16
        return 0

    lax.fori_loop(0, K, zrow, 0)
    for b in range(N_PER_T // K):
        pltpu.sync_copy(rows0, acc_sh.at[pl.ds(sid * N_PER_T + b * K, K)])

    pltpu.sync_copy(c_hbm.at[wid], cf_v)
    pltpu.sync_copy(s_hbm.at[wid, pl.ds(0, 2)], sidx0)
    pltpu.sync_copy(s_hbm.at[wid, pl.ds(0, 2)], sidx1)
    pltpu.sync_copy(r_hbm.at[wid, pl.ds(0, 2)], ridx0)
    pltpu.sync_copy(r_hbm.at[wid, pl.ds(0, 2)], ridx1)
    plsc.subcore_barrier()

    GS = 4
    KS = K // GS

    def fire_gather(idx_buf, row, rows_ref, sem):
        for p in range(GS):
            pltpu.async_copy(nf_hbm.at[idx_buf.at[row, pl.ds(p * KS, KS)]],
                             rows_ref.at[pl.ds(p * KS, KS)], sem)

    def wait_gather(idx_buf, row, rows_ref, sem):
        for p in range(GS):
            pltpu.make_async_copy(nf_hbm.at[idx_buf.at[row, pl.ds(p * KS, KS)]],
                                  rows_ref.at[pl.ds(p * KS, KS)], sem).wait()

    def scale(rows_ref, jj):
        def grp(g, _):
            cfv = cf_v[jj, pl.ds(g * 16, 16)]
            for l in range(16):
                c16 = jnp.full((16,), cfv[l])
                r = g * 16 + l
                for t in range(C // 16):
                    rows_ref[r, pl.ds(t * 16, 16)] = rows_ref[r, pl.ds(t * 16, 16)] * c16
            return 0
        lax.fori_loop(0, K // 16, grp, 0)

    fire_gather(sidx0, 0, rows0, gsem0)
    fire_gather(sidx1, 1, rows1, gsem1)

    def body(j2, _):
        a = 2 * j2
        last = CH // 2 - 1

        wait_gather(sidx0, 0, rows0, gsem0)

        @pl.when(j2 < last)
        def _():
            pltpu.async_copy(s_hbm.at[wid, pl.ds(a + 2, 2)], sidx0, sisem0)

        scale(rows0, a)

        @pl.when(j2 > 0)
        def _():
            pltpu.make_async_copy(r_hbm.at[wid, pl.ds(a, 2)], ridx0, risem0).wait()

        pltpu.async_copy(rows0, acc_sh.at[ridx0.at[0]], ssem0, add=True)

        wait_gather(sidx1, 1, rows1, gsem1)

        @pl.when(j2 < last)
        def _():
            pltpu.async_copy(s_hbm.at[wid, pl.ds(a + 2, 2)], sidx1, sisem1)

        scale(rows1, a + 1)

        @pl.when(j2 > 0)
        def _():
            pltpu.make_async_copy(r_hbm.at[wid, pl.ds(a, 2)], ridx1, risem1).wait()

        pltpu.async_copy(rows1, acc_sh.at[ridx1.at[1]], ssem1, add=True)

        pltpu.make_async_copy(rows0, acc_sh.at[ridx0.at[0]], ssem0).wait()

        @pl.when(j2 < last)
        def _():
            pltpu.async_copy(r_hbm.at[wid, pl.ds(a + 2, 2)], ridx0, risem0)
            pltpu.make_async_copy(s_hbm.at[wid, pl.ds(a + 2, 2)], sidx0, sisem0).wait()
            fire_gather(sidx0, 0, rows0, gsem0)

        pltpu.make_async_copy(rows1, acc_sh.at[ridx1.at[1]], ssem1).wait()

        @pl.when(j2 < last)
        def _():
            pltpu.async_copy(r_hbm.at[wid, pl.ds(a + 2, 2)], ridx1, risem1)
            pltpu.make_async_copy(s_hbm.at[wid, pl.ds(a + 2, 2)], sidx1, sisem1).wait()
            fire_gather(sidx1, 1, rows1, gsem1)

        return 0

    lax.fori_loop(0, CH // 2, body, 0)
    plsc.subcore_barrier()

    pltpu.sync_copy(acc_sh.at[pl.ds(sid * N_PER_T, N_PER_T)],
                    out_hbm.at[cid, pl.ds(sid * N_PER_T, N_PER_T)])


def _scatter_stage(nf, senders, receivers, coeff):
    pad = E_PAD - E
    s3 = jnp.concatenate([senders, jnp.zeros((pad,), jnp.int32)]).reshape(NW, CH, K)
    r3 = jnp.concatenate([receivers, jnp.zeros((pad,), jnp.int32)]).reshape(NW, CH, K)
    c3 = coeff.reshape(NW, CH, K)
    mesh = plsc.VectorSubcoreMesh(core_axis_name="c", subcore_axis_name="s")
    f = pl.kernel(
        _sc_body,
        mesh=mesh,
        out_type=jax.ShapeDtypeStruct((NC, N_PAD, C), jnp.float32),
        scratch_types=[
            pltpu.VMEM((2, K), jnp.int32),
            pltpu.VMEM((2, K), jnp.int32),
            pltpu.VMEM((2, K), jnp.int32),
            pltpu.VMEM((2, K), jnp.int32),
            pltpu.VMEM((CH, K), jnp.float32),
            pltpu.VMEM((K, C), jnp.float32),
            pltpu.VMEM((K, C), jnp.float32),
            pltpu.VMEM_SHARED((N_PAD, C), jnp.float32),
        ] + [pltpu.SemaphoreType.DMA] * 8,
    )
    return f(nf, s3, r3, c3)


def _final_body(acc_ref, w2_ref, out_ref):
    acc = acc_ref[0] + acc_ref[1]
    out_ref[...] = jnp.dot(acc, w2_ref[...], preferred_element_type=jnp.float32) * _inv_sqrt_C


def _final_stage(acc, W2):
    BN = 400
    grid = (N // BN,)
    return pl.pallas_call(
        _final_body,
        grid=grid,
        in_specs=[
            pl.BlockSpec((NC, BN, C), lambda i: (0, i, 0)),
            pl.BlockSpec((C, H), lambda i: (0, 0)),
        ],
        out_specs=pl.BlockSpec((BN, H), lambda i: (i, 0)),
        out_shape=jax.ShapeDtypeStruct((N, H), jnp.float32),
    )(acc, W2)


@jax.jit
def kernel(node_attrs, node_feats, edge_attrs, edge_feats, senders, receivers,
           W_sc, W1, Wm1, Wm2, Wm3, Wm4, W2):
    nf, sc = _node_stage(node_feats, node_attrs, W1, W_sc)
    coeff = _edge_stage(edge_feats, edge_attrs, Wm1, Wm2, Wm3, Wm4)
    acc = _scatter_stage(nf, senders, receivers, coeff)
    message = _final_stage(acc, W2)
    return (message, sc)


# speedup vs baseline: 1.3003x; 1.3003x over previous
import functools
import numpy as np
import jax
import jax.numpy as jnp
from jax import lax
from jax.experimental import pallas as pl
from jax.experimental.pallas import tpu as pltpu
from jax.experimental.pallas import tpu_sc as plsc

N = 10000
E = 320000
C = 128
A = 4
S = 4
R = 8
H = 128
AVG_NUM_NEIGHBORS = 32.0
MLP_HIDDEN = 64

NC = 2
NS = 16
NW = NC * NS

CH2 = C // 2
K = 112
CH = 184
E_PER_T = CH * K
E_PAD = NS * E_PER_T

_inv_sqrt_C = np.float32(1.0 / np.sqrt(C))
_inv_sqrt_CA = np.float32(1.0 / np.sqrt(C * A))
_inv_sqrt_R = np.float32(1.0 / np.sqrt(R))
_inv_sqrt_M = np.float32(1.0 / np.sqrt(MLP_HIDDEN))
_coeff_scale = np.float32(1.0 / (np.sqrt(S) * np.sqrt(AVG_NUM_NEIGHBORS)))


def _silu(x):
    return x / (1.0 + jnp.exp(-x))


def _node_body(f_ref, attr_ref, w1_ref, wsc_ref, nf_ref, sc_ref):
    f = f_ref[...]
    nf_ref[...] = jnp.dot(f, w1_ref[...], preferred_element_type=jnp.float32) * _inv_sqrt_C
    acc = jnp.zeros_like(sc_ref)
    for a in range(A):
        fa = f * attr_ref[:, a][:, None]
        acc = acc + jnp.dot(fa, wsc_ref[a], preferred_element_type=jnp.float32)
    sc_ref[...] = acc * _inv_sqrt_CA


def _node_stage(node_feats, node_attrs, W1, W_sc):
    BN = 400
    grid = (N // BN,)
    wsc = W_sc.reshape(C, A, H).transpose(1, 0, 2)
    return pl.pallas_call(
        _node_body,
        grid=grid,
        in_specs=[
            pl.BlockSpec((BN, C), lambda i: (i, 0)),
            pl.BlockSpec((BN, A), lambda i: (i, 0)),
            pl.BlockSpec((C, C), lambda i: (0, 0)),
            pl.BlockSpec((A, C, H), lambda i: (0, 0, 0)),
        ],
        out_specs=[
            pl.BlockSpec((BN, C), lambda i: (i, 0)),
            pl.BlockSpec((BN, H), lambda i: (i, 0)),
        ],
        out_shape=[
            jax.ShapeDtypeStruct((N, C), jnp.float32),
            jax.ShapeDtypeStruct((N, H), jnp.float32),
        ],
    )(node_feats, node_attrs, W1, wsc)


def _edge_body(ef_ref, ea_ref, w1_ref, w2_ref, w3_ref, w4_ref, coeff_ref):
    h = _silu(jnp.dot(w1_ref[...], ef_ref[...], preferred_element_type=jnp.float32) * _inv_sqrt_R)
    h = _silu(jnp.dot(w2_ref[...], h, preferred_element_type=jnp.float32) * _inv_sqrt_M)
    h = _silu(jnp.dot(w3_ref[...], h, preferred_element_type=jnp.float32) * _inv_sqrt_M)
    lw = jnp.dot(w4_ref[...], h, preferred_element_type=jnp.float32) * _inv_sqrt_M
    coeff_ref[...] = jnp.sum(lw * ea_ref[...], axis=0) * _coeff_scale


def _edge_stage(edge_feats, edge_attrs, Wm1, Wm2, Wm3, Wm4):
    BE = 2048
    eft = jnp.zeros((R, E_PAD), jnp.float32).at[:, :E].set(edge_feats.T)
    eat = jnp.zeros((S, E_PAD), jnp.float32).at[:, :E].set(edge_attrs.T)
    grid = (E_PAD // BE,)
    return pl.pallas_call(
        _edge_body,
        grid=grid,
        in_specs=[
            pl.BlockSpec((R, BE), lambda i: (0, i)),
            pl.BlockSpec((S, BE), lambda i: (0, i)),
            pl.BlockSpec((MLP_HIDDEN, R), lambda i: (0, 0)),
            pl.BlockSpec((MLP_HIDDEN, MLP_HIDDEN), lambda i: (0, 0)),
            pl.BlockSpec((MLP_HIDDEN, MLP_HIDDEN), lambda i: (0, 0)),
            pl.BlockSpec((S, MLP_HIDDEN), lambda i: (0, 0)),
        ],
        out_specs=pl.BlockSpec((BE,), lambda i: (i,)),
        out_shape=jax.ShapeDtypeStruct((E_PAD,), jnp.float32),
    )(eft, eat, Wm1.T, Wm2.T, Wm3.T, Wm4.T)


N_PAD = 10240
N_PER_T = N_PAD // NS


def _sc_body(tab_hbm, s_hbm, r_hbm, c_hbm, out_hbm,
             sidx0, sidx1, ridx0, ridx1, cf0, cf1, raw0, raw1, sh,
             gsem0, gsem1, ssem0, ssem1, sisem0, sisem1,
             risem0, risem1, csem0, csem1):
    cid = lax.axis_index("c")
    sid = lax.axis_index("s")

    pltpu.sync_copy(tab_hbm.at[cid, pl.ds(sid * N_PER_T, N_PER_T)],
                    sh.at[pl.ds(sid * N_PER_T, N_PER_T)])

    pltpu.sync_copy(s_hbm.at[sid, pl.ds(0, 2)], sidx0)
    pltpu.sync_copy(s_hbm.at[sid, pl.ds(0, 2)], sidx1)
    pltpu.sync_copy(r_hbm.at[sid, pl.ds(0, 2)], ridx0)
    pltpu.sync_copy(r_hbm.at[sid, pl.ds(0, 2)], ridx1)
    pltpu.sync_copy(c_hbm.at[sid, pl.ds(0, 2)], cf0)
    pltpu.sync_copy(c_hbm.at[sid, pl.ds(0, 2)], cf1)
    plsc.subcore_barrier()

    def scale(raw_ref, cf_buf, row):
        def grp(g, _):
            cfv = cf_buf[row, pl.ds(g * 16, 16)]
            z16 = jnp.zeros((16,), jnp.float32)
            for l in range(16):
                c16 = jnp.full((16,), cfv[l])
                r = g * 16 + l
                for t in range(CH2 // 16):
                    x = raw_ref[r, pl.ds(t * 16, 16)]
                    raw_ref[r, pl.ds(t * 16, 16)] = z16
                    raw_ref[r, pl.ds(CH2 + t * 16, 16)] = x * c16
            return 0
        lax.fori_loop(0, K // 16, grp, 0)

    pltpu.async_copy(sh.at[sidx0.at[0]], raw0, gsem0)
    pltpu.async_copy(sh.at[sidx1.at[1]], raw1, gsem1)

    def body(j2, _):
        a = 2 * j2
        last = CH // 2 - 1

        pltpu.make_async_copy(sh.at[sidx0.at[0]], raw0, gsem0).wait()

        @pl.when(j2 < last)
        def _():
            pltpu.async_copy(s_hbm.at[sid, pl.ds(a + 2, 2)], sidx0, sisem0)

        @pl.when(j2 > 0)
        def _():
            pltpu.make_async_copy(c_hbm.at[sid, pl.ds(a, 2)], cf0, csem0).wait()

        scale(raw0, cf0, 0)

        @pl.when(j2 > 0)
        def _():
            pltpu.make_async_copy(r_hbm.at[sid, pl.ds(a, 2)], ridx0, risem0).wait()

        pltpu.async_copy(raw0, sh.at[ridx0.at[0]], ssem0, add=True)

        @pl.when(j2 < last)
        def _():
            pltpu.async_copy(c_hbm.at[sid, pl.ds(a + 2, 2)], cf0, csem0)

        pltpu.make_async_copy(sh.at[sidx1.at[1]], raw1, gsem1).wait()

        @pl.when(j2 < last)
        def _():
            pltpu.async_copy(s_hbm.at[sid, pl.ds(a + 2, 2)], sidx1, sisem1)

        @pl.when(j2 > 0)
        def _():
            pltpu.make_async_copy(c_hbm.at[sid, pl.ds(a, 2)], cf1, csem1).wait()

        scale(raw1, cf1, 1)

        @pl.when(j2 > 0)
        def _():
            pltpu.make_async_copy(r_hbm.at[sid, pl.ds(a, 2)], ridx1, risem1).wait()

        pltpu.async_copy(raw1, sh.at[ridx1.at[1]], ssem1, add=True)

        @pl.when(j2 < last)
        def _():
            pltpu.async_copy(c_hbm.at[sid, pl.ds(a + 2, 2)], cf1, csem1)

        pltpu.make_async_copy(raw0, sh.at[ridx0.at[0]], ssem0).wait()

        @pl.when(j2 < last)
        def _():
            pltpu.async_copy(r_hbm.at[sid, pl.ds(a + 2, 2)], ridx0, risem0)
            pltpu.make_async_copy(s_hbm.at[sid, pl.ds(a + 2, 2)], sidx0, sisem0).wait()
            pltpu.async_copy(sh.at[sidx0.at[0]], raw0, gsem0)

        pltpu.make_async_copy(raw1, sh.at[ridx1.at[1]], ssem1).wait()

        @pl.when(j2 < last)
        def _():
            pltpu.async_copy(r_hbm.at[sid, pl.ds(a + 2, 2)], ridx1, risem1)
            pltpu.make_async_copy(s_hbm.at[sid, pl.ds(a + 2, 2)], sidx1, sisem1).wait()
            pltpu.async_copy(sh.at[sidx1.at[1]], raw1, gsem1)

        return 0

    lax.fori_loop(0, CH // 2, body, 0)
    plsc.subcore_barrier()

    pltpu.sync_copy(sh.at[pl.ds(sid * N_PER_T, N_PER_T)],
                    out_hbm.at[cid, pl.ds(sid * N_PER_T, N_PER_T)])


def _scatter_stage(nf, senders, receivers, coeff):
    pad = E_PAD - E
    s3 = jnp.concatenate([senders, jnp.zeros((pad,), jnp.int32)]).reshape(NS, CH, K)
    r3 = jnp.concatenate([receivers, jnp.zeros((pad,), jnp.int32)]).reshape(NS, CH, K)
    c3 = coeff.reshape(NS, CH, K)
    tab = jnp.zeros((NC, N_PAD, C), jnp.float32)
    tab = tab.at[0, :N, :CH2].set(nf[:, :CH2])
    tab = tab.at[1, :N, :CH2].set(nf[:, CH2:])
    mesh = plsc.VectorSubcoreMesh(core_axis_name="c", subcore_axis_name="s")
    f = pl.kernel(
        _sc_body,
        mesh=mesh,
        out_type=jax.ShapeDtypeStruct((NC, N_PAD, C), jnp.float32),
        scratch_types=[
            pltpu.VMEM((2, K), jnp.int32),
            pltpu.VMEM((2, K), jnp.int32),
            pltpu.VMEM((2, K), jnp.int32),
            pltpu.VMEM((2, K), jnp.int32),
            pltpu.VMEM((2, K), jnp.float32),
            pltpu.VMEM((2, K), jnp.float32),
            pltpu.VMEM((K, C), jnp.float32),
            pltpu.VMEM((K, C), jnp.float32),
            pltpu.VMEM_SHARED((N_PAD, C), jnp.float32),
        ] + [pltpu.SemaphoreType.DMA] * 10,
    )
    return f(tab, s3, r3, c3)


def _final_body(acc_ref, w2_ref, out_ref):
    lo = acc_ref[0][:, CH2:]
    hi = acc_ref[1][:, CH2:]
    out = jnp.dot(lo, w2_ref[pl.ds(0, CH2), :], preferred_element_type=jnp.float32)
    out = out + jnp.dot(hi, w2_ref[pl.ds(CH2, CH2), :], preferred_element_type=jnp.float32)
    out_ref[...] = out * _inv_sqrt_C


def _final_stage(acc, W2):
    BN = 400
    grid = (N // BN,)
    return pl.pallas_call(
        _final_body,
        grid=grid,
        in_specs=[
            pl.BlockSpec((NC, BN, C), lambda i: (0, i, 0)),
            pl.BlockSpec((C, H), lambda i: (0, 0)),
        ],
        out_specs=pl.BlockSpec((BN, H), lambda i: (i, 0)),
        out_shape=jax.ShapeDtypeStruct((N, H), jnp.float32),
    )(acc, W2)


@jax.jit
def kernel(node_attrs, node_feats, edge_attrs, edge_feats, senders, receivers,
           W_sc, W1, Wm1, Wm2, Wm3, Wm4, W2):
    nf, sc = _node_stage(node_feats, node_attrs, W1, W_sc)
    coeff = _edge_stage(edge_feats, edge_attrs, Wm1, Wm2, Wm3, Wm4)
    acc = _scatter_stage(nf, senders, receivers, coeff)
    message = _final_stage(acc, W2)
    return (message, sc)
